# baseline (speedup 1.0000x reference)
import functools

import jax
import jax.numpy as jnp
from jax import lax
from jax.experimental import pallas as pl
from jax.experimental.pallas import tpu as pltpu
from jax.experimental.pallas import tpu_sc as plsc

B = 4096
L = 200
D = 32
NC = 2
NS = 16
NW = NC * NS
BB = B // 128
LA = L // 8

_mesh = plsc.VectorSubcoreMesh(core_axis_name="c", subcore_axis_name="s")


@functools.partial(
    pl.kernel,
    out_type=jax.ShapeDtypeStruct((L, D // 8, BB, 8, 128), jnp.float32),
    mesh=_mesh,
    scratch_types=[
        pltpu.VMEM((LA, 8, 128), jnp.int32),
        pltpu.VMEM((128, D), jnp.float32),
        pltpu.VMEM((128, D), jnp.float32),
        pltpu.VMEM((D // 8, 8, 128), jnp.float32),
        pltpu.VMEM((D // 8, 8, 128), jnp.float32),
        pltpu.VMEM((L * D,), jnp.float32),
        pltpu.SemaphoreType.DMA,
        pltpu.SemaphoreType.DMA,
        pltpu.SemaphoreType.DMA,
        pltpu.SemaphoreType.DMA,
    ],
    compiler_params=pltpu.CompilerParams(
        use_tc_tiling_on_sc=False, needs_layout_passes=False
    ),
)
def _sc_embed(idx_hbm, table_hbm, pos_hbm, out_hbm,
              idxcol, gbuf0, gbuf1, sout0, sout1, posv,
              gsem0, gsem1, ssem0, ssem1):
    w = lax.axis_index("s") * NC + lax.axis_index("c")
    pltpu.sync_copy(pos_hbm, posv)
    pltpu.sync_copy(idx_hbm.at[:, w], idxcol)

    gbufs = [gbuf0, gbuf1]
    souts = [sout0, sout1]
    gsems = [gsem0, gsem1]
    ssems = [ssem0, ssem1]
    iota = jnp.arange(16, dtype=jnp.int32)
    rowidx = [iota + 16 * g for g in range(8)]

    def fire_gather(l, q):
        pltpu.async_copy(
            table_hbm.at[idxcol.at[l >> 3, l & 7]], gbufs[q], gsems[q]
        )

    fire_gather(jnp.int32(0), 0)

    def step(l, p, store_drain):
        q = p ^ 1
        nxt = l + 1

        @pl.when(nxt < L)
        def _():
            fire_gather(nxt, q)

        pltpu.make_async_copy(table_hbm.at[idxcol.at[0, 0]], gbufs[p], gsems[p]).wait()

        @pl.when(store_drain)
        def _():
            pltpu.make_async_copy(out_hbm.at[0, :, 0], souts[p], ssems[p]).wait()

        gb, so = gbufs[p], souts[p]
        for d in range(D):
            pos_bc = plsc.load_gather(posv, [jnp.broadcast_to(l + 200 * d, (16,))])
            cold = jnp.broadcast_to(jnp.int32(d), (16,))
            for g in range(8):
                v = plsc.load_gather(gb, [rowidx[g], cold]) + pos_bc
                so[d // 8, d % 8, pl.ds(16 * g, 16)] = v
        pltpu.async_copy(so, out_hbm.at[l, :, w], ssems[p])

    def body(j, _):
        step(2 * j, 0, j >= 1)
        step(2 * j + 1, 1, j >= 1)
        return 0

    lax.fori_loop(0, L // 2, body, 0)
    pltpu.make_async_copy(out_hbm.at[0, :, 0], sout0, ssem0).wait()
    pltpu.make_async_copy(out_hbm.at[0, :, 0], sout1, ssem1).wait()


def kernel(inputs, word_table, pos_table):
    idx4 = (
        inputs.T.astype(jnp.int32)
        .reshape(LA, 8, BB, 128)
        .transpose(0, 2, 1, 3)
    )
    posflat = pos_table.T.reshape(-1)
    x = _sc_embed(idx4, word_table, posflat)
    return x.transpose(2, 4, 0, 1, 3).reshape(B, L, D)


# speedup vs baseline: 1.4942x; 1.4942x over previous
import functools

import jax
import jax.numpy as jnp
from jax import lax
from jax.experimental import pallas as pl
from jax.experimental.pallas import tpu as pltpu
from jax.experimental.pallas import tpu_sc as plsc

B = 4096
L = 200
D = 32
NC = 2
NS = 16
NW = NC * NS
BB = B // 128
LA = L // 8
PITCH = 133

_mesh = plsc.VectorSubcoreMesh(core_axis_name="c", subcore_axis_name="s")


@functools.partial(
    pl.kernel,
    out_type=jax.ShapeDtypeStruct((L, D // 8, BB, 8, 128), jnp.float32),
    mesh=_mesh,
    scratch_types=[
        pltpu.VMEM((LA, 8, 128), jnp.int32),
        pltpu.VMEM((128, D), jnp.float32),
        pltpu.VMEM((128, D), jnp.float32),
        pltpu.VMEM((D // 8, 8, PITCH), jnp.float32),
        pltpu.VMEM((D // 8, 8, PITCH), jnp.float32),
        pltpu.VMEM((L, D), jnp.float32),
        pltpu.SemaphoreType.DMA,
        pltpu.SemaphoreType.DMA,
        pltpu.SemaphoreType.DMA,
        pltpu.SemaphoreType.DMA,
    ],
    compiler_params=pltpu.CompilerParams(
        use_tc_tiling_on_sc=False, needs_layout_passes=False
    ),
)
def _sc_embed(idx_hbm, table_hbm, pos_hbm, out_hbm,
              idxcol, gbuf0, gbuf1, sout0, sout1, posv,
              gsem0, gsem1, ssem0, ssem1):
    w = lax.axis_index("s") * NC + lax.axis_index("c")
    pltpu.sync_copy(pos_hbm, posv)
    pltpu.sync_copy(idx_hbm.at[:, w], idxcol)

    gbufs = [gbuf0, gbuf1]
    souts = [sout0, sout1]
    gsems = [gsem0, gsem1]
    ssems = [ssem0, ssem1]
    iota = jnp.arange(16, dtype=jnp.int32)
    r_lo, u_lo = iota // 8, iota % 8
    r_hi, u_hi = (iota + 16) // 8, iota % 8

    def fire_gather(l, q):
        pltpu.async_copy(
            table_hbm.at[idxcol.at[l >> 3, l & 7]], gbufs[q], gsems[q]
        )

    fire_gather(jnp.int32(0), 0)

    def step(l, p, store_drain):
        q = p ^ 1
        nxt = l + 1

        @pl.when(nxt < L)
        def _():
            fire_gather(nxt, q)

        pltpu.make_async_copy(table_hbm.at[idxcol.at[0, 0]], gbufs[p], gsems[p]).wait()

        @pl.when(store_drain)
        def _():
            pltpu.make_async_copy(
                out_hbm.at[0, :, 0], souts[p].at[:, :, pl.ds(0, 128)], ssems[p]
            ).wait()

        gb, so = gbufs[p], souts[p]
        pos_lo = posv[l, pl.ds(0, 16)]
        pos_hi = posv[l, pl.ds(16, 16)]
        for b in range(128):
            col = jnp.broadcast_to(jnp.int32(b), (16,))
            plsc.store_scatter(so, [r_lo, u_lo, col], gb[b, pl.ds(0, 16)] + pos_lo)
            plsc.store_scatter(so, [r_hi, u_hi, col], gb[b, pl.ds(16, 16)] + pos_hi)
        pltpu.async_copy(
            so.at[:, :, pl.ds(0, 128)], out_hbm.at[l, :, w], ssems[p]
        )

    def body(j, _):
        step(2 * j, 0, j >= 1)
        step(2 * j + 1, 1, j >= 1)
        return 0

    lax.fori_loop(0, L // 2, body, 0)
    pltpu.make_async_copy(out_hbm.at[0, :, 0], sout0.at[:, :, pl.ds(0, 128)], ssem0).wait()
    pltpu.make_async_copy(out_hbm.at[0, :, 0], sout1.at[:, :, pl.ds(0, 128)], ssem1).wait()


def kernel(inputs, word_table, pos_table):
    idx4 = (
        inputs.T.astype(jnp.int32)
        .reshape(LA, 8, BB, 128)
        .transpose(0, 2, 1, 3)
    )
    x = _sc_embed(idx4, word_table, pos_table)
    return x.transpose(2, 4, 0, 1, 3).reshape(B, L, D)
